# baseline (speedup 1.0000x reference)
import functools

import jax
import jax.numpy as jnp
from jax import lax
from jax.experimental import pallas as pl
from jax.experimental.pallas import tpu as pltpu

IN_F = 4096
HID = 64
OUT_F = 1
BATCH = 8192
BM = 512
NCHUNK = 4


def _mlp_body(x_hbm, w1_ref, w2_ref, w3_ref, o_ref, xbuf, sems):
    i = pl.program_id(0)
    n = pl.num_programs(0)
    rows = BM // NCHUNK

    def dma(step, slot, c):
        return pltpu.make_async_copy(
            x_hbm.at[pl.ds(step * BM + c * rows, rows), :],
            xbuf.at[slot, pl.ds(c * rows, rows), :],
            sems.at[slot, c],
        )

    def start(step, slot):
        for c in range(NCHUNK):
            dma(step, slot, c).start()

    def wait(step, slot):
        for c in range(NCHUNK):
            dma(step, slot, c).wait()

    @pl.when(i == 0)
    def _():
        start(0, 0)

    slot = lax.rem(i, 2)

    @pl.when(i + 1 < n)
    def _():
        start(i + 1, lax.rem(i + 1, 2))

    wait(i, slot)
    o_ref[...] = jnp.sum(xbuf[slot], axis=1, keepdims=True)


@jax.jit
def _fused_mlp(x, w1, w2, w3):
    return pl.pallas_call(
        _mlp_body,
        grid=(BATCH // BM,),
        in_specs=[
            pl.BlockSpec(memory_space=pl.ANY),
            pl.BlockSpec((IN_F, HID), lambda i: (0, 0)),
            pl.BlockSpec((HID, HID), lambda i: (0, 0)),
            pl.BlockSpec((HID, OUT_F), lambda i: (0, 0)),
        ],
        out_specs=pl.BlockSpec((BM, OUT_F), lambda i: (i, 0)),
        out_shape=jax.ShapeDtypeStruct((BATCH, OUT_F), jnp.float32),
        scratch_shapes=[
            pltpu.VMEM((2, BM, IN_F), jnp.float32),
            pltpu.SemaphoreType.DMA((2, NCHUNK)),
        ],
    )(x, w1, w2, w3)


def kernel(x, idx1, val1, idx2, val2, idx3, val3):
    x = x.reshape(x.shape[0], -1)
    w1 = jnp.zeros((IN_F, HID), jnp.float32).at[idx1[0], idx1[1]].add(val1)
    w2 = jnp.zeros((HID, HID), jnp.float32).at[idx2[0], idx2[1]].add(val2)
    w3 = jnp.zeros((HID, OUT_F), jnp.float32).at[idx3[0], idx3[1]].add(val3)
    return _fused_mlp(x, w1, w2, w3)


# speedup vs baseline: 2.2322x; 2.2322x over previous
import functools

import jax
import jax.numpy as jnp
from jax import lax
from jax.experimental import pallas as pl
from jax.experimental.pallas import tpu as pltpu
from jax.experimental.pallas import tpu_sc as plsc

IN_F = 4096
HID = 64
OUT_F = 1
BATCH = 8192
BM = 512

_SC_INFO = plsc.get_sparse_core_info()
_NC = _SC_INFO.num_cores
_NS = _SC_INFO.num_subcores
_L = _SC_INFO.num_lanes
_NW = _NC * _NS

W1_SLOTS = IN_F * HID
W2_SLOTS = HID * HID
W3_SLOTS = HID * OUT_F
S1 = W1_SLOTS // _NW
S2 = W2_SLOTS // _NW
_U = 4


def _padded(n):
    m = _L * _U
    return (n + m - 1) // m * m


@functools.lru_cache(maxsize=4)
def _make_densify(p1, p2, p3):
    mesh = plsc.VectorSubcoreMesh(core_axis_name="c", subcore_axis_name="s")

    def body(c1, v1, c2, v2, c3, v3, w1o, w2o, w3o,
             c1v, v1v, c2v, v2v, c3v, v3v, slab1, slab2, slab3, sem1, sem2):
        wid = lax.axis_index("s") * _NC + lax.axis_index("c")
        cp1 = pltpu.async_copy(c1, c1v, sem1)
        cp2 = pltpu.async_copy(v1, v1v, sem2)

        zeros = jnp.zeros((_L,), jnp.float32)

        def zbody(i, carry):
            slab1[pl.ds(i * _L, _L)] = zeros
            return carry

        lax.fori_loop(0, S1 // _L, zbody, 0)
        for i in range(S2 // _L):
            slab2[pl.ds(i * _L, _L)] = zeros
        for i in range(W3_SLOTS // _L):
            slab3[pl.ds(i * _L, _L)] = zeros

        pltpu.sync_copy(c2, c2v)
        pltpu.sync_copy(v2, v2v)
        pltpu.sync_copy(c3, c3v)
        pltpu.sync_copy(v3, v3v)
        cp1.wait()
        cp2.wait()

        lo1 = wid * S1

        def body1(i, carry):
            for u in range(_U):
                base = (i * _U + u) * _L
                cc = c1v[pl.ds(base, _L)]
                vv = v1v[pl.ds(base, _L)]
                loc = cc - lo1
                m = (loc >= 0) & (loc < S1)
                loc = jnp.where(m, loc, 0)
                plsc.addupdate_scatter(slab1, [loc], vv, mask=m)
            return carry

        lax.fori_loop(0, p1 // (_L * _U), body1, 0)

        lo2 = wid * S2
        for i in range(p2 // _L):
            cc = c2v[pl.ds(i * _L, _L)]
            vv = v2v[pl.ds(i * _L, _L)]
            loc = cc - lo2
            m = (loc >= 0) & (loc < S2)
            loc = jnp.where(m, loc, 0)
            plsc.addupdate_scatter(slab2, [loc], vv, mask=m)

        @pl.when(wid == 0)
        def _():
            for i in range(p3 // _L):
                cc = c3v[pl.ds(i * _L, _L)]
                vv = v3v[pl.ds(i * _L, _L)]
                m = (cc >= 0) & (cc < W3_SLOTS)
                loc = jnp.where(m, cc, 0)
                plsc.addupdate_scatter(slab3, [loc], vv, mask=m)

        pltpu.sync_copy(slab1, w1o.at[pl.ds(lo1, S1)])
        pltpu.sync_copy(slab2, w2o.at[pl.ds(lo2, S2)])

        @pl.when(wid == 0)
        def _():
            pltpu.sync_copy(slab3, w3o)

    return pl.kernel(
        body,
        mesh=mesh,
        compiler_params=pltpu.CompilerParams(needs_layout_passes=False),
        out_type=[
            jax.ShapeDtypeStruct((W1_SLOTS,), jnp.float32),
            jax.ShapeDtypeStruct((W2_SLOTS,), jnp.float32),
            jax.ShapeDtypeStruct((W3_SLOTS,), jnp.float32),
        ],
        scratch_types=[
            pltpu.VMEM((p1,), jnp.int32),
            pltpu.VMEM((p1,), jnp.float32),
            pltpu.VMEM((p2,), jnp.int32),
            pltpu.VMEM((p2,), jnp.float32),
            pltpu.VMEM((p3,), jnp.int32),
            pltpu.VMEM((p3,), jnp.float32),
            pltpu.VMEM((S1,), jnp.float32),
            pltpu.VMEM((S2,), jnp.float32),
            pltpu.VMEM((W3_SLOTS,), jnp.float32),
            pltpu.SemaphoreType.DMA,
            pltpu.SemaphoreType.DMA,
        ],
    )


def _pack(idx, val, out_dim, padded):
    n = val.shape[0]
    combo = idx[0] * out_dim + idx[1]
    combo = jnp.pad(combo, (0, padded - n))
    val = jnp.pad(val, (0, padded - n))
    return combo.astype(jnp.int32), val


def _mlp_body(x_ref, w1_ref, w2_ref, w3_ref, o_ref):
    h = jnp.maximum(
        jnp.dot(x_ref[...], w1_ref[...], preferred_element_type=jnp.float32), 0.0
    )
    h = jnp.maximum(
        jnp.dot(h, w2_ref[...], preferred_element_type=jnp.float32), 0.0
    )
    o_ref[...] = jnp.dot(h, w3_ref[...], preferred_element_type=jnp.float32)


def _fused_mlp(x, w1, w2, w3):
    return pl.pallas_call(
        _mlp_body,
        grid=(BATCH // BM,),
        in_specs=[
            pl.BlockSpec((BM, IN_F), lambda i: (i, 0)),
            pl.BlockSpec((IN_F, HID), lambda i: (0, 0)),
            pl.BlockSpec((HID, HID), lambda i: (0, 0)),
            pl.BlockSpec((HID, OUT_F), lambda i: (0, 0)),
        ],
        out_specs=pl.BlockSpec((BM, OUT_F), lambda i: (i, 0)),
        out_shape=jax.ShapeDtypeStruct((BATCH, OUT_F), jnp.float32),
    )(x, w1, w2, w3)


def kernel(x, idx1, val1, idx2, val2, idx3, val3):
    x = x.reshape(x.shape[0], -1)
    p1 = _padded(val1.shape[0])
    p2 = _padded(val2.shape[0])
    p3 = _padded(val3.shape[0])
    c1, v1 = _pack(idx1, val1, HID, p1)
    c2, v2 = _pack(idx2, val2, HID, p2)
    c3, v3 = _pack(idx3, val3, OUT_F, p3)
    w1f, w2f, w3f = _make_densify(p1, p2, p3)(c1, v1, c2, v2, c3, v3)
    w1 = w1f.reshape(IN_F, HID)
    w2 = w2f.reshape(HID, HID)
    w3 = w3f.reshape(HID, OUT_F)
    return _fused_mlp(x, w1, w2, w3)


# speedup vs baseline: 2.3251x; 1.0416x over previous
import functools

import jax
import jax.numpy as jnp
from jax import lax
from jax.experimental import pallas as pl
from jax.experimental.pallas import tpu as pltpu
from jax.experimental.pallas import tpu_sc as plsc

IN_F = 4096
HID = 64
OUT_F = 1
BATCH = 8192
BM = 512

_SC_INFO = plsc.get_sparse_core_info()
_NC = _SC_INFO.num_cores
_NS = _SC_INFO.num_subcores
_L = _SC_INFO.num_lanes
_NW = _NC * _NS

W1_SLOTS = IN_F * HID
W2_SLOTS = HID * HID
W3_SLOTS = HID * OUT_F
_T1 = 2
_R1 = _NW // _T1
S1 = W1_SLOTS // _R1
S2 = W2_SLOTS // _NW
_U = 4


def _padded(n):
    m = _L * _U * _T1
    return (n + m - 1) // m * m


@functools.lru_cache(maxsize=4)
def _make_densify(p1, p2, p3):
    mesh = plsc.VectorSubcoreMesh(core_axis_name="c", subcore_axis_name="s")

    def body(c1, v1, c2, v2, c3, v3, w1o, w2o, w3o,
             c1v, v1v, c2v, v2v, c3v, v3v, slab1, slab2, slab3, sem1, sem2):
        wid = lax.axis_index("s") * _NC + lax.axis_index("c")
        rng = wid // _T1
        half = wid % _T1
        hn = p1 // _T1
        cp1 = pltpu.async_copy(c1.at[pl.ds(half * hn, hn)], c1v, sem1)
        cp2 = pltpu.async_copy(v1.at[pl.ds(half * hn, hn)], v1v, sem2)

        zeros = jnp.zeros((_L,), jnp.float32)

        def zbody(i, carry):
            slab1[pl.ds(i * _L, _L)] = zeros
            return carry

        lax.fori_loop(0, S1 // _L, zbody, 0)
        for i in range(S2 // _L):
            slab2[pl.ds(i * _L, _L)] = zeros
        for i in range(W3_SLOTS // _L):
            slab3[pl.ds(i * _L, _L)] = zeros

        pltpu.sync_copy(c2, c2v)
        pltpu.sync_copy(v2, v2v)
        pltpu.sync_copy(c3, c3v)
        pltpu.sync_copy(v3, v3v)
        cp1.wait()
        cp2.wait()

        lo1 = rng * S1

        def body1(i, carry):
            for u in range(_U):
                base = (i * _U + u) * _L
                cc = c1v[pl.ds(base, _L)]
                vv = v1v[pl.ds(base, _L)]
                loc = cc - lo1
                m = (loc >= 0) & (loc < S1)
                loc = jnp.where(m, loc, 0)
                plsc.addupdate_scatter(slab1, [loc], vv, mask=m)
            return carry

        lax.fori_loop(0, hn // (_L * _U), body1, 0)

        lo2 = wid * S2
        for i in range(p2 // _L):
            cc = c2v[pl.ds(i * _L, _L)]
            vv = v2v[pl.ds(i * _L, _L)]
            loc = cc - lo2
            m = (loc >= 0) & (loc < S2)
            loc = jnp.where(m, loc, 0)
            plsc.addupdate_scatter(slab2, [loc], vv, mask=m)

        @pl.when(wid == 0)
        def _():
            for i in range(p3 // _L):
                cc = c3v[pl.ds(i * _L, _L)]
                vv = v3v[pl.ds(i * _L, _L)]
                m = (cc >= 0) & (cc < W3_SLOTS)
                loc = jnp.where(m, cc, 0)
                plsc.addupdate_scatter(slab3, [loc], vv, mask=m)

        pltpu.sync_copy(slab1, w1o.at[pl.ds(half * W1_SLOTS + lo1, S1)])
        pltpu.sync_copy(slab2, w2o.at[pl.ds(lo2, S2)])

        @pl.when(wid == 0)
        def _():
            pltpu.sync_copy(slab3, w3o)

    return pl.kernel(
        body,
        mesh=mesh,
        compiler_params=pltpu.CompilerParams(needs_layout_passes=False),
        out_type=[
            jax.ShapeDtypeStruct((_T1 * W1_SLOTS,), jnp.float32),
            jax.ShapeDtypeStruct((W2_SLOTS,), jnp.float32),
            jax.ShapeDtypeStruct((W3_SLOTS,), jnp.float32),
        ],
        scratch_types=[
            pltpu.VMEM((p1 // _T1,), jnp.int32),
            pltpu.VMEM((p1 // _T1,), jnp.float32),
            pltpu.VMEM((p2,), jnp.int32),
            pltpu.VMEM((p2,), jnp.float32),
            pltpu.VMEM((p3,), jnp.int32),
            pltpu.VMEM((p3,), jnp.float32),
            pltpu.VMEM((S1,), jnp.float32),
            pltpu.VMEM((S2,), jnp.float32),
            pltpu.VMEM((W3_SLOTS,), jnp.float32),
            pltpu.SemaphoreType.DMA,
            pltpu.SemaphoreType.DMA,
        ],
    )


def _pack(idx, val, out_dim, padded):
    n = val.shape[0]
    combo = idx[0] * out_dim + idx[1]
    combo = jnp.pad(combo, (0, padded - n))
    val = jnp.pad(val, (0, padded - n))
    return combo.astype(jnp.int32), val


def _mlp_body(x_ref, w1_ref, w2_ref, w3_ref, o_ref):
    w1 = w1_ref[0] + w1_ref[1]
    h = jnp.maximum(
        jnp.dot(x_ref[...], w1, preferred_element_type=jnp.float32), 0.0
    )
    h = jnp.maximum(
        jnp.dot(h, w2_ref[...], preferred_element_type=jnp.float32), 0.0
    )
    o_ref[...] = jnp.dot(h, w3_ref[...], preferred_element_type=jnp.float32)


def _fused_mlp(x, w1, w2, w3):
    return pl.pallas_call(
        _mlp_body,
        grid=(BATCH // BM,),
        in_specs=[
            pl.BlockSpec((BM, IN_F), lambda i: (i, 0)),
            pl.BlockSpec((_T1, IN_F, HID), lambda i: (0, 0, 0)),
            pl.BlockSpec((HID, HID), lambda i: (0, 0)),
            pl.BlockSpec((HID, OUT_F), lambda i: (0, 0)),
        ],
        out_specs=pl.BlockSpec((BM, OUT_F), lambda i: (i, 0)),
        out_shape=jax.ShapeDtypeStruct((BATCH, OUT_F), jnp.float32),
    )(x, w1, w2, w3)


def kernel(x, idx1, val1, idx2, val2, idx3, val3):
    x = x.reshape(x.shape[0], -1)
    p1 = _padded(val1.shape[0])
    p2 = _padded(val2.shape[0])
    p3 = _padded(val3.shape[0])
    c1, v1 = _pack(idx1, val1, HID, p1)
    c2, v2 = _pack(idx2, val2, HID, p2)
    c3, v3 = _pack(idx3, val3, OUT_F, p3)
    w1f, w2f, w3f = _make_densify(p1, p2, p3)(c1, v1, c2, v2, c3, v3)
    w1 = w1f.reshape(_T1, IN_F, HID)
    w2 = w2f.reshape(HID, HID)
    w3 = w3f.reshape(HID, OUT_F)
    return _fused_mlp(x, w1, w2, w3)


# speedup vs baseline: 2.3339x; 1.0038x over previous
import functools

import jax
import jax.numpy as jnp
from jax import lax
from jax.experimental import pallas as pl
from jax.experimental.pallas import tpu as pltpu
from jax.experimental.pallas import tpu_sc as plsc

IN_F = 4096
HID = 64
OUT_F = 1
BATCH = 8192
BM = 1024

_SC_INFO = plsc.get_sparse_core_info()
_NC = _SC_INFO.num_cores
_NS = _SC_INFO.num_subcores
_L = _SC_INFO.num_lanes
_NW = _NC * _NS

W1_SLOTS = IN_F * HID
W2_SLOTS = HID * HID
W3_SLOTS = HID * OUT_F
_T1 = 2
_R1 = _NW // _T1
S1 = W1_SLOTS // _R1
S2 = W2_SLOTS // _NW
_U = 4


def _padded(n):
    m = _L * _U * _T1
    return (n + m - 1) // m * m


@functools.lru_cache(maxsize=4)
def _make_densify(p1, p2, p3):
    mesh = plsc.VectorSubcoreMesh(core_axis_name="c", subcore_axis_name="s")

    def body(c1, v1, c2, v2, c3, v3, w1o, w2o, w3o,
             c1v, v1v, c2v, v2v, c3v, v3v, slab1, slab2, slab3, sem1, sem2):
        wid = lax.axis_index("s") * _NC + lax.axis_index("c")
        rng = wid // _T1
        half = wid % _T1
        hn = p1 // _T1
        cp1 = pltpu.async_copy(c1.at[pl.ds(half * hn, hn)], c1v, sem1)
        cp2 = pltpu.async_copy(v1.at[pl.ds(half * hn, hn)], v1v, sem2)

        zeros = jnp.zeros((_L,), jnp.float32)

        def zbody(i, carry):
            slab1[pl.ds(i * _L, _L)] = zeros
            return carry

        lax.fori_loop(0, S1 // _L, zbody, 0)
        for i in range(S2 // _L):
            slab2[pl.ds(i * _L, _L)] = zeros
        for i in range(W3_SLOTS // _L):
            slab3[pl.ds(i * _L, _L)] = zeros

        pltpu.sync_copy(c2, c2v)
        pltpu.sync_copy(v2, v2v)
        pltpu.sync_copy(c3, c3v)
        pltpu.sync_copy(v3, v3v)
        cp1.wait()
        cp2.wait()

        lo1 = rng * S1

        def body1(i, carry):
            for u in range(_U):
                base = (i * _U + u) * _L
                cc = c1v[pl.ds(base, _L)]
                vv = v1v[pl.ds(base, _L)]
                loc = cc - lo1
                m = (loc >= 0) & (loc < S1)
                loc = jnp.where(m, loc, 0)
                plsc.addupdate_scatter(slab1, [loc], vv, mask=m)
            return carry

        lax.fori_loop(0, hn // (_L * _U), body1, 0)

        lo2 = wid * S2
        for i in range(p2 // _L):
            cc = c2v[pl.ds(i * _L, _L)]
            vv = v2v[pl.ds(i * _L, _L)]
            loc = cc - lo2
            m = (loc >= 0) & (loc < S2)
            loc = jnp.where(m, loc, 0)
            plsc.addupdate_scatter(slab2, [loc], vv, mask=m)

        @pl.when(wid == 0)
        def _():
            for i in range(p3 // _L):
                cc = c3v[pl.ds(i * _L, _L)]
                vv = v3v[pl.ds(i * _L, _L)]
                m = (cc >= 0) & (cc < W3_SLOTS)
                loc = jnp.where(m, cc, 0)
                plsc.addupdate_scatter(slab3, [loc], vv, mask=m)

        pltpu.sync_copy(slab1, w1o.at[pl.ds(half * W1_SLOTS + lo1, S1)])
        pltpu.sync_copy(slab2, w2o.at[pl.ds(lo2, S2)])

        @pl.when(wid == 0)
        def _():
            pltpu.sync_copy(slab3, w3o)

    return pl.kernel(
        body,
        mesh=mesh,
        compiler_params=pltpu.CompilerParams(needs_layout_passes=False),
        out_type=[
            jax.ShapeDtypeStruct((_T1 * W1_SLOTS,), jnp.float32),
            jax.ShapeDtypeStruct((W2_SLOTS,), jnp.float32),
            jax.ShapeDtypeStruct((W3_SLOTS,), jnp.float32),
        ],
        scratch_types=[
            pltpu.VMEM((p1 // _T1,), jnp.int32),
            pltpu.VMEM((p1 // _T1,), jnp.float32),
            pltpu.VMEM((p2,), jnp.int32),
            pltpu.VMEM((p2,), jnp.float32),
            pltpu.VMEM((p3,), jnp.int32),
            pltpu.VMEM((p3,), jnp.float32),
            pltpu.VMEM((S1,), jnp.float32),
            pltpu.VMEM((S2,), jnp.float32),
            pltpu.VMEM((W3_SLOTS,), jnp.float32),
            pltpu.SemaphoreType.DMA,
            pltpu.SemaphoreType.DMA,
        ],
    )


def _pack(idx, val, out_dim, padded):
    n = val.shape[0]
    combo = idx[0] * out_dim + idx[1]
    combo = jnp.pad(combo, (0, padded - n))
    val = jnp.pad(val, (0, padded - n))
    return combo.astype(jnp.int32), val


def _mlp_body(x_ref, w1_ref, w2_ref, w3_ref, o_ref):
    w1 = w1_ref[0] + w1_ref[1]
    h = jnp.maximum(
        jnp.dot(x_ref[...], w1, preferred_element_type=jnp.float32), 0.0
    )
    h = jnp.maximum(
        jnp.dot(h, w2_ref[...], preferred_element_type=jnp.float32), 0.0
    )
    o_ref[...] = jnp.dot(h, w3_ref[...], preferred_element_type=jnp.float32)


def _fused_mlp(x, w1, w2, w3):
    return pl.pallas_call(
        _mlp_body,
        grid=(BATCH // BM,),
        in_specs=[
            pl.BlockSpec((BM, IN_F), lambda i: (i, 0)),
            pl.BlockSpec((_T1, IN_F, HID), lambda i: (0, 0, 0)),
            pl.BlockSpec((HID, HID), lambda i: (0, 0)),
            pl.BlockSpec((HID, OUT_F), lambda i: (0, 0)),
        ],
        out_specs=pl.BlockSpec((BM, OUT_F), lambda i: (i, 0)),
        out_shape=jax.ShapeDtypeStruct((BATCH, OUT_F), jnp.float32),
    )(x, w1, w2, w3)


def kernel(x, idx1, val1, idx2, val2, idx3, val3):
    x = x.reshape(x.shape[0], -1)
    p1 = _padded(val1.shape[0])
    p2 = _padded(val2.shape[0])
    p3 = _padded(val3.shape[0])
    c1, v1 = _pack(idx1, val1, HID, p1)
    c2, v2 = _pack(idx2, val2, HID, p2)
    c3, v3 = _pack(idx3, val3, OUT_F, p3)
    w1f, w2f, w3f = _make_densify(p1, p2, p3)(c1, v1, c2, v2, c3, v3)
    w1 = w1f.reshape(_T1, IN_F, HID)
    w2 = w2f.reshape(HID, HID)
    w3 = w3f.reshape(HID, OUT_F)
    return _fused_mlp(x, w1, w2, w3)


# speedup vs baseline: 2.3805x; 1.0200x over previous
import functools

import jax
import jax.numpy as jnp
from jax import lax
from jax.experimental import pallas as pl
from jax.experimental.pallas import tpu as pltpu
from jax.experimental.pallas import tpu_sc as plsc

IN_F = 4096
HID = 64
OUT_F = 1
BATCH = 8192
BM = 1024

_SC_INFO = plsc.get_sparse_core_info()
_NC = _SC_INFO.num_cores
_NS = _SC_INFO.num_subcores
_L = _SC_INFO.num_lanes
_NW = _NC * _NS

W1_SLOTS = IN_F * HID
W2_SLOTS = HID * HID
W3_SLOTS = HID * OUT_F
_T1 = 4
_R1 = _NW // _T1
S1 = W1_SLOTS // _R1
S2 = W2_SLOTS // _NW
_U = 4


def _padded(n):
    m = _L * _U * _T1
    return (n + m - 1) // m * m


@functools.lru_cache(maxsize=4)
def _make_densify(p1, p2, p3):
    mesh = plsc.VectorSubcoreMesh(core_axis_name="c", subcore_axis_name="s")

    def body(c1, v1, c2, v2, c3, v3, w1o, w2o, w3o,
             c1v, v1v, c2v, v2v, c3v, v3v, slab1, slab2, slab3, sem1, sem2):
        wid = lax.axis_index("s") * _NC + lax.axis_index("c")
        rng = wid // _T1
        half = wid % _T1
        hn = p1 // _T1
        cp1 = pltpu.async_copy(c1.at[pl.ds(half * hn, hn)], c1v, sem1)
        cp2 = pltpu.async_copy(v1.at[pl.ds(half * hn, hn)], v1v, sem2)

        zeros = jnp.zeros((_L,), jnp.float32)
        _ZU = 8

        def zbody(i, carry):
            for z in range(_ZU):
                slab1[pl.ds((i * _ZU + z) * _L, _L)] = zeros
            return carry

        lax.fori_loop(0, S1 // (_L * _ZU), zbody, 0)
        for i in range(S2 // _L):
            slab2[pl.ds(i * _L, _L)] = zeros
        for i in range(W3_SLOTS // _L):
            slab3[pl.ds(i * _L, _L)] = zeros

        pltpu.sync_copy(c2, c2v)
        pltpu.sync_copy(v2, v2v)
        pltpu.sync_copy(c3, c3v)
        pltpu.sync_copy(v3, v3v)
        cp1.wait()
        cp2.wait()

        lo1 = rng * S1

        def body1(i, carry):
            for u in range(_U):
                base = (i * _U + u) * _L
                cc = c1v[pl.ds(base, _L)]
                vv = v1v[pl.ds(base, _L)]
                loc = cc - lo1
                m = (loc >= 0) & (loc < S1)
                loc = jnp.where(m, loc, 0)
                plsc.addupdate_scatter(slab1, [loc], vv, mask=m)
            return carry

        lax.fori_loop(0, hn // (_L * _U), body1, 0)

        lo2 = wid * S2
        for i in range(p2 // _L):
            cc = c2v[pl.ds(i * _L, _L)]
            vv = v2v[pl.ds(i * _L, _L)]
            loc = cc - lo2
            m = (loc >= 0) & (loc < S2)
            loc = jnp.where(m, loc, 0)
            plsc.addupdate_scatter(slab2, [loc], vv, mask=m)

        @pl.when(wid == 0)
        def _():
            for i in range(p3 // _L):
                cc = c3v[pl.ds(i * _L, _L)]
                vv = v3v[pl.ds(i * _L, _L)]
                m = (cc >= 0) & (cc < W3_SLOTS)
                loc = jnp.where(m, cc, 0)
                plsc.addupdate_scatter(slab3, [loc], vv, mask=m)

        pltpu.sync_copy(slab1, w1o.at[pl.ds(half * W1_SLOTS + lo1, S1)])
        pltpu.sync_copy(slab2, w2o.at[pl.ds(lo2, S2)])

        @pl.when(wid == 0)
        def _():
            pltpu.sync_copy(slab3, w3o)

    return pl.kernel(
        body,
        mesh=mesh,
        compiler_params=pltpu.CompilerParams(needs_layout_passes=False),
        out_type=[
            jax.ShapeDtypeStruct((_T1 * W1_SLOTS,), jnp.float32),
            jax.ShapeDtypeStruct((W2_SLOTS,), jnp.float32),
            jax.ShapeDtypeStruct((W3_SLOTS,), jnp.float32),
        ],
        scratch_types=[
            pltpu.VMEM((p1 // _T1,), jnp.int32),
            pltpu.VMEM((p1 // _T1,), jnp.float32),
            pltpu.VMEM((p2,), jnp.int32),
            pltpu.VMEM((p2,), jnp.float32),
            pltpu.VMEM((p3,), jnp.int32),
            pltpu.VMEM((p3,), jnp.float32),
            pltpu.VMEM((S1,), jnp.float32),
            pltpu.VMEM((S2,), jnp.float32),
            pltpu.VMEM((W3_SLOTS,), jnp.float32),
            pltpu.SemaphoreType.DMA,
            pltpu.SemaphoreType.DMA,
        ],
    )


def _pack(idx, val, out_dim, padded):
    n = val.shape[0]
    combo = idx[0] * out_dim + idx[1]
    combo = jnp.pad(combo, (0, padded - n))
    val = jnp.pad(val, (0, padded - n))
    return combo.astype(jnp.int32), val


def _mlp_body(x_ref, w1_ref, w2_ref, w3_ref, o_ref):
    w1 = w1_ref[0]
    for t in range(1, _T1):
        w1 = w1 + w1_ref[t]
    h = jnp.maximum(
        jnp.dot(x_ref[...], w1, preferred_element_type=jnp.float32), 0.0
    )
    h = jnp.maximum(
        jnp.dot(h, w2_ref[...], preferred_element_type=jnp.float32), 0.0
    )
    o_ref[...] = jnp.dot(h, w3_ref[...], preferred_element_type=jnp.float32)


def _fused_mlp(x, w1, w2, w3):
    return pl.pallas_call(
        _mlp_body,
        grid=(BATCH // BM,),
        in_specs=[
            pl.BlockSpec((BM, IN_F), lambda i: (i, 0)),
            pl.BlockSpec((_T1, IN_F, HID), lambda i: (0, 0, 0)),
            pl.BlockSpec((HID, HID), lambda i: (0, 0)),
            pl.BlockSpec((HID, OUT_F), lambda i: (0, 0)),
        ],
        out_specs=pl.BlockSpec((BM, OUT_F), lambda i: (i, 0)),
        out_shape=jax.ShapeDtypeStruct((BATCH, OUT_F), jnp.float32),
    )(x, w1, w2, w3)


def kernel(x, idx1, val1, idx2, val2, idx3, val3):
    x = x.reshape(x.shape[0], -1)
    p1 = _padded(val1.shape[0])
    p2 = _padded(val2.shape[0])
    p3 = _padded(val3.shape[0])
    c1, v1 = _pack(idx1, val1, HID, p1)
    c2, v2 = _pack(idx2, val2, HID, p2)
    c3, v3 = _pack(idx3, val3, OUT_F, p3)
    w1f, w2f, w3f = _make_densify(p1, p2, p3)(c1, v1, c2, v2, c3, v3)
    w1 = w1f.reshape(_T1, IN_F, HID)
    w2 = w2f.reshape(HID, HID)
    w3 = w3f.reshape(HID, OUT_F)
    return _fused_mlp(x, w1, w2, w3)
